# Initial kernel scaffold; baseline (speedup 1.0000x reference)
import jax
import jax.numpy as jnp
from jax.experimental import pallas as pl


def kernel(x, edge_index, W, b):
    raise NotImplementedError("write your pallas kernel here")


# speedup vs baseline: 36.8117x; 36.8117x over previous
import functools

import jax
import jax.numpy as jnp
from jax import lax
from jax.experimental import pallas as pl
from jax.experimental.pallas import tpu as pltpu
from jax.experimental.pallas import tpu_sc as plsc

N = 10000
D = 128
NC = 2
NS = 16
NW = NC * NS
CH = 128
NPAD = N + 112
RPT = NPAD // NS
RB = 2000


def _mesh():
    return plsc.VectorSubcoreMesh(core_axis_name="c", subcore_axis_name="s")


def _deg_count(row2):
    nch = row2.shape[0] // NW

    @functools.partial(
        pl.kernel,
        mesh=_mesh(),
        out_type=jax.ShapeDtypeStruct((NC, NPAD, 16), jnp.float32),
        scratch_types=[
            pltpu.VMEM_SHARED((NPAD, 16), jnp.float32),
            pltpu.VMEM((CH, 16), jnp.float32),
            pltpu.VMEM((RPT, 16), jnp.float32),
            pltpu.VMEM((CH,), jnp.int32),
            pltpu.VMEM((CH,), jnp.int32),
            pltpu.SemaphoreType.DMA,
            pltpu.SemaphoreType.DMA,
        ],
    )
    def deg_kernel(row_hbm, deg_hbm, deg_sh, ones_v, zero_v, iv0, iv1, s0, s1):
        cid = lax.axis_index("c")
        sid = lax.axis_index("s")
        wid = cid * NS + sid
        cb = wid * nch

        def fill(i, carry):
            ones_v[i] = jnp.full((16,), 1.0, jnp.float32)
            return carry

        lax.fori_loop(0, CH, fill, 0)

        def zfill(i, carry):
            zero_v[i] = jnp.zeros((16,), jnp.float32)
            return carry

        lax.fori_loop(0, RPT, zfill, 0)
        pltpu.sync_copy(zero_v, deg_sh.at[pl.ds(sid * RPT, RPT)])
        plsc.subcore_barrier()

        pltpu.async_copy(row_hbm.at[cb + 0], iv0, s0)
        pltpu.async_copy(row_hbm.at[cb + 1], iv1, s1)
        slots = ((iv0, s0), (iv1, s1))

        def body(c2, carry):
            for b in range(2):
                c = c2 * 2 + b
                iv, s = slots[b]
                pltpu.make_async_copy(row_hbm.at[cb + c], iv, s).wait()
                pltpu.sync_copy(ones_v, deg_sh.at[iv], add=True)

                @pl.when(c + 2 < nch)
                def _():
                    pltpu.async_copy(row_hbm.at[cb + c + 2], iv, s)

            return carry

        lax.fori_loop(0, nch // 2, body, 0)
        plsc.subcore_barrier()
        pltpu.sync_copy(
            deg_sh.at[pl.ds(sid * RPT, RPT)],
            deg_hbm.at[cid, pl.ds(sid * RPT, RPT)],
        )

    return deg_kernel(row2)


def _edge_scatter(g, row2, col2):
    nch = row2.shape[0] // NW

    @functools.partial(
        pl.kernel,
        mesh=_mesh(),
        out_type=jax.ShapeDtypeStruct((NC, NPAD, D), jnp.float32),
        scratch_types=[
            pltpu.VMEM_SHARED((NPAD, D), jnp.float32),
            pltpu.VMEM((CH, D), jnp.float32),
            pltpu.VMEM((CH, D), jnp.float32),
            pltpu.VMEM((CH,), jnp.int32),
            pltpu.VMEM((CH,), jnp.int32),
            pltpu.VMEM((CH,), jnp.int32),
            pltpu.VMEM((CH,), jnp.int32),
            pltpu.SemaphoreType.DMA,
            pltpu.SemaphoreType.DMA,
            pltpu.SemaphoreType.DMA,
            pltpu.SemaphoreType.DMA,
        ],
    )
    def scat_kernel(g_hbm, row_hbm, col_hbm, acc_hbm, acc_sh,
                    gb0, gb1, rv0, rv1, cv0, cv1, i0, i1, gs0, gs1):
        cid = lax.axis_index("c")
        sid = lax.axis_index("s")
        wid = cid * NS + sid
        cb = wid * nch

        def z(i, carry):
            for j in range(D // 16):
                gb0[i, pl.ds(j * 16, 16)] = jnp.zeros((16,), jnp.float32)
            return carry

        lax.fori_loop(0, CH, z, 0)
        r0 = sid * RPT
        for k in range(RPT // CH):
            pltpu.sync_copy(gb0, acc_sh.at[pl.ds(r0 + k * CH, CH)])
        rem = RPT % CH
        if rem:
            pltpu.sync_copy(
                gb0.at[pl.ds(0, rem)],
                acc_sh.at[pl.ds(r0 + (RPT // CH) * CH, rem)],
            )
        plsc.subcore_barrier()

        pltpu.async_copy(row_hbm.at[cb + 0], rv0, i0)
        pltpu.async_copy(col_hbm.at[cb + 0], cv0, i0)
        pltpu.async_copy(row_hbm.at[cb + 1], rv1, i1)
        pltpu.async_copy(col_hbm.at[cb + 1], cv1, i1)
        pltpu.make_async_copy(row_hbm.at[cb + 0], rv0, i0).wait()
        pltpu.make_async_copy(col_hbm.at[cb + 0], cv0, i0).wait()
        pltpu.async_copy(g_hbm.at[cv0], gb0, gs0)

        slots = ((rv0, cv0, gb0, i0, gs0), (rv1, cv1, gb1, i1, gs1))

        def body(c2, carry):
            for b in range(2):
                c = c2 * 2 + b
                rv, cv, gb, isem, gsem = slots[b]
                rv_n, cv_n, gb_n, isem_n, gsem_n = slots[1 - b]

                @pl.when(c + 1 < nch)
                def _():
                    pltpu.make_async_copy(row_hbm.at[cb + c + 1], rv_n, isem_n).wait()
                    pltpu.make_async_copy(col_hbm.at[cb + c + 1], cv_n, isem_n).wait()
                    pltpu.async_copy(g_hbm.at[cv_n], gb_n, gsem_n)

                pltpu.make_async_copy(g_hbm.at[cv], gb, gsem).wait()
                pltpu.sync_copy(gb, acc_sh.at[rv], add=True)

                @pl.when(c + 2 < nch)
                def _():
                    pltpu.async_copy(row_hbm.at[cb + c + 2], rv, isem)
                    pltpu.async_copy(col_hbm.at[cb + c + 2], cv, isem)

            return carry

        lax.fori_loop(0, nch // 2, body, 0)
        plsc.subcore_barrier()
        pltpu.sync_copy(
            acc_sh.at[pl.ds(r0, RPT)],
            acc_hbm.at[cid, pl.ds(r0, RPT)],
        )

    return scat_kernel(g, row2, col2)


def _lin_body(deg_ref, x_ref, w_ref, b_ref, g_ref):
    d = deg_ref[0, :, 0:1] + deg_ref[1, :, 0:1] + 1.0
    dis = lax.rsqrt(d + 1e-12)
    h = jnp.dot(x_ref[...], w_ref[...], preferred_element_type=jnp.float32)
    g_ref[...] = (h + b_ref[...]) * dis


def _fused_linear(degp, x, W, b2):
    return pl.pallas_call(
        _lin_body,
        grid=(N // RB,),
        in_specs=[
            pl.BlockSpec((NC, RB, 16), lambda i: (0, i, 0)),
            pl.BlockSpec((RB, D), lambda i: (i, 0)),
            pl.BlockSpec((D, D), lambda i: (0, 0)),
            pl.BlockSpec((1, D), lambda i: (0, 0)),
        ],
        out_specs=pl.BlockSpec((RB, D), lambda i: (i, 0)),
        out_shape=jax.ShapeDtypeStruct((N, D), jnp.float32),
    )(degp, x, W, b2)


def _comb_body(deg_ref, acc_ref, g_ref, out_ref):
    d = deg_ref[0, :, 0:1] + deg_ref[1, :, 0:1] + 1.0
    dis = lax.rsqrt(d + 1e-12)
    s = acc_ref[0] + acc_ref[1] + g_ref[...]
    out_ref[...] = s * dis


def _combine(degp, accp, g):
    return pl.pallas_call(
        _comb_body,
        grid=(N // RB,),
        in_specs=[
            pl.BlockSpec((NC, RB, 16), lambda i: (0, i, 0)),
            pl.BlockSpec((NC, RB, D), lambda i: (0, i, 0)),
            pl.BlockSpec((RB, D), lambda i: (i, 0)),
        ],
        out_specs=pl.BlockSpec((RB, D), lambda i: (i, 0)),
        out_shape=jax.ShapeDtypeStruct((N, D), jnp.float32),
    )(degp, accp, g)


def kernel(x, edge_index, W, b):
    E = edge_index.shape[1]
    unit = NW * CH * 2
    e_pad = ((E + unit - 1) // unit) * unit
    pad = e_pad - E
    row = edge_index[0]
    col = edge_index[1]
    ar = jnp.arange(pad, dtype=jnp.int32)
    row_p = jnp.concatenate([row, N + (ar % 16)])
    col_p = jnp.concatenate([col, (ar * 2003) % N])
    row2 = row_p.reshape(-1, CH)
    col2 = col_p.reshape(-1, CH)

    degp = _deg_count(row2)
    g = _fused_linear(degp, x, W, b.reshape(1, D))
    accp = _edge_scatter(g, row2, col2)
    return _combine(degp, accp, g)
